# baseline (speedup 1.0000x reference)
import jax
import jax.numpy as jnp
from jax.experimental import pallas as pl
from jax.experimental.pallas import tpu as pltpu

_NUM_EMB = 1024
_EMB_DIM = 256
_COMMIT = 0.25
_TS = 2048
_NC = 128
_NCHUNK = _NUM_EMB // _NC


def _vq_tile(z_ref, cb2_ref, cbt_hi_ref, cbt_lo_ref,
             zq_ref, idx_ref, loss_ref, csq_ref):
    first = (pl.program_id(0) == 0) & (pl.program_id(1) == 0)

    @pl.when(first)
    def _():
        cb2 = cb2_ref[...]
        csq = 0.25 * jnp.sum(cb2 * cb2, axis=1, keepdims=True)
        csq_ref[...] = jnp.broadcast_to(csq, (_NUM_EMB, _TS))
        loss_ref[...] = jnp.zeros_like(loss_ref)

    zb = z_ref[0]
    z_sq = jnp.sum(zb * zb, axis=0, keepdims=True)

    minv = None
    idx = None
    for k in range(_NCHUNK):
        sl = pl.ds(k * _NC, _NC)
        d2 = jax.lax.dot_general(
            cb2_ref[sl, :], zb, (((1,), (0,)), ((), ())),
            preferred_element_type=jnp.float32)
        s = (csq_ref[sl, :] + z_sq) + d2
        mk = jnp.min(s, axis=0, keepdims=True)
        row = jax.lax.broadcasted_iota(jnp.int32, s.shape, 0) + k * _NC
        ik = jnp.min(jnp.where(s == mk, row, _NUM_EMB),
                     axis=0, keepdims=True)
        if k == 0:
            minv, idx = mk, ik
        else:
            take = mk < minv
            idx = jnp.where(take, ik, idx)
            minv = jnp.minimum(minv, mk)

    zq = jnp.zeros((_EMB_DIM, _TS), jnp.float32)
    dn = (((1,), (0,)), ((), ()))
    for k in range(_NCHUNK):
        row = jax.lax.broadcasted_iota(jnp.int32, (_NC, _TS), 0) + k * _NC
        oh = (row == idx).astype(jnp.bfloat16)
        sl = pl.ds(k * _NC, _NC)
        zq = zq + jax.lax.dot_general(cbt_hi_ref[:, sl], oh, dn,
                                      preferred_element_type=jnp.float32)

    zq_ref[0] = zb + (zq - zb)
    idx_ref[0] = idx
    loss_ref[...] += jnp.sum(minv, axis=1, keepdims=True)


def kernel(z_e, codebook):
    B, C, D, H, W = z_e.shape
    S = D * H * W
    z = z_e.reshape(B, C, S)

    cb2 = -2.0 * codebook
    cbt = codebook.T
    cbt_hi = cbt.astype(jnp.bfloat16)
    cbt_lo = (cbt - cbt_hi.astype(jnp.float32)).astype(jnp.bfloat16)

    zq, idx, loss = pl.pallas_call(
        _vq_tile,
        grid=(B, S // _TS),
        in_specs=[
            pl.BlockSpec((1, C, _TS), lambda b, t: (b, 0, t)),
            pl.BlockSpec((_NUM_EMB, _EMB_DIM), lambda b, t: (0, 0)),
            pl.BlockSpec((_EMB_DIM, _NUM_EMB), lambda b, t: (0, 0)),
            pl.BlockSpec((_EMB_DIM, _NUM_EMB), lambda b, t: (0, 0)),
        ],
        out_specs=[
            pl.BlockSpec((1, C, _TS), lambda b, t: (b, 0, t)),
            pl.BlockSpec((1, 1, _TS), lambda b, t: (b, 0, t)),
            pl.BlockSpec((1, 1), lambda b, t: (0, 0)),
        ],
        out_shape=[
            jax.ShapeDtypeStruct((B, C, S), jnp.float32),
            jax.ShapeDtypeStruct((B, 1, S), jnp.int32),
            jax.ShapeDtypeStruct((1, 1), jnp.float32),
        ],
        scratch_shapes=[pltpu.VMEM((_NUM_EMB, _TS), jnp.float32)],
    )(z, cb2, cbt_hi, cbt_lo)

    m = loss[0, 0] / jnp.float32(B * S * C)
    vq_loss = m + jnp.float32(_COMMIT) * m
    return (zq.reshape(B, C, D, H, W), vq_loss,
            idx.reshape(B, D, H, W))


# speedup vs baseline: 1.1653x; 1.1653x over previous
import jax
import jax.numpy as jnp
from jax.experimental import pallas as pl
from jax.experimental.pallas import tpu as pltpu

_NUM_EMB = 1024
_EMB_DIM = 256
_COMMIT = 0.25
_TS = 2048
_NC = 1024
_NCHUNK = _NUM_EMB // _NC


def _vq_tile(z_ref, cb2_ref, cbt_hi_ref, cbt_lo_ref,
             zq_ref, idx_ref, loss_ref, csq_ref):
    first = (pl.program_id(0) == 0) & (pl.program_id(1) == 0)

    @pl.when(first)
    def _():
        cb2 = cb2_ref[...]
        csq = 0.25 * jnp.sum(cb2 * cb2, axis=1, keepdims=True)
        csq_ref[...] = jnp.broadcast_to(csq, (_NUM_EMB, _TS))
        loss_ref[...] = jnp.zeros_like(loss_ref)

    zb = z_ref[0]
    z_sq = jnp.sum(zb * zb, axis=0, keepdims=True)

    minv = None
    idx = None
    for k in range(_NCHUNK):
        sl = pl.ds(k * _NC, _NC)
        d2 = jax.lax.dot_general(
            cb2_ref[sl, :], zb, (((1,), (0,)), ((), ())),
            preferred_element_type=jnp.float32)
        s = (csq_ref[sl, :] + z_sq) + d2
        mk = jnp.min(s, axis=0, keepdims=True)
        row = jax.lax.broadcasted_iota(jnp.int32, s.shape, 0) + k * _NC
        ik = jnp.min(jnp.where(s == mk, row, _NUM_EMB),
                     axis=0, keepdims=True)
        if k == 0:
            minv, idx = mk, ik
        else:
            take = mk < minv
            idx = jnp.where(take, ik, idx)
            minv = jnp.minimum(minv, mk)

    zq = jnp.zeros((_EMB_DIM, _TS), jnp.float32)
    dn = (((1,), (0,)), ((), ()))
    for k in range(_NCHUNK):
        row = jax.lax.broadcasted_iota(jnp.int32, (_NC, _TS), 0) + k * _NC
        oh = (row == idx).astype(jnp.bfloat16)
        sl = pl.ds(k * _NC, _NC)
        zq = zq + jax.lax.dot_general(cbt_hi_ref[:, sl], oh, dn,
                                      preferred_element_type=jnp.float32)

    zq_ref[0] = zb + (zq - zb)
    idx_ref[0] = idx
    loss_ref[...] += jnp.sum(minv, axis=1, keepdims=True)


def kernel(z_e, codebook):
    B, C, D, H, W = z_e.shape
    S = D * H * W
    z = z_e.reshape(B, C, S)

    cb2 = -2.0 * codebook
    cbt = codebook.T
    cbt_hi = cbt.astype(jnp.bfloat16)
    cbt_lo = (cbt - cbt_hi.astype(jnp.float32)).astype(jnp.bfloat16)

    zq, idx, loss = pl.pallas_call(
        _vq_tile,
        grid=(B, S // _TS),
        in_specs=[
            pl.BlockSpec((1, C, _TS), lambda b, t: (b, 0, t)),
            pl.BlockSpec((_NUM_EMB, _EMB_DIM), lambda b, t: (0, 0)),
            pl.BlockSpec((_EMB_DIM, _NUM_EMB), lambda b, t: (0, 0)),
            pl.BlockSpec((_EMB_DIM, _NUM_EMB), lambda b, t: (0, 0)),
        ],
        out_specs=[
            pl.BlockSpec((1, C, _TS), lambda b, t: (b, 0, t)),
            pl.BlockSpec((1, 1, _TS), lambda b, t: (b, 0, t)),
            pl.BlockSpec((1, 1), lambda b, t: (0, 0)),
        ],
        out_shape=[
            jax.ShapeDtypeStruct((B, C, S), jnp.float32),
            jax.ShapeDtypeStruct((B, 1, S), jnp.int32),
            jax.ShapeDtypeStruct((1, 1), jnp.float32),
        ],
        scratch_shapes=[pltpu.VMEM((_NUM_EMB, _TS), jnp.float32)],
    )(z, cb2, cbt_hi, cbt_lo)

    m = loss[0, 0] / jnp.float32(B * S * C)
    vq_loss = m + jnp.float32(_COMMIT) * m
    return (zq.reshape(B, C, D, H, W), vq_loss,
            idx.reshape(B, D, H, W))


# speedup vs baseline: 1.1698x; 1.0039x over previous
import jax
import jax.numpy as jnp
from jax.experimental import pallas as pl
from jax.experimental.pallas import tpu as pltpu

_NUM_EMB = 1024
_EMB_DIM = 256
_COMMIT = 0.25
_TS = 4096
_NC = 1024
_NCHUNK = _NUM_EMB // _NC


def _vq_tile(z_ref, cb2_ref, cbt_hi_ref, cbt_lo_ref,
             zq_ref, idx_ref, loss_ref, csq_ref):
    first = (pl.program_id(0) == 0) & (pl.program_id(1) == 0)

    @pl.when(first)
    def _():
        cb2 = cb2_ref[...]
        csq = 0.25 * jnp.sum(cb2 * cb2, axis=1, keepdims=True)
        csq_ref[...] = jnp.broadcast_to(csq, (_NUM_EMB, _TS))
        loss_ref[...] = jnp.zeros_like(loss_ref)

    zb = z_ref[0]
    z_sq = jnp.sum(zb * zb, axis=0, keepdims=True)

    minv = None
    idx = None
    for k in range(_NCHUNK):
        sl = pl.ds(k * _NC, _NC)
        d2 = jax.lax.dot_general(
            cb2_ref[sl, :], zb, (((1,), (0,)), ((), ())),
            preferred_element_type=jnp.float32)
        s = (csq_ref[sl, :] + z_sq) + d2
        mk = jnp.min(s, axis=0, keepdims=True)
        row = jax.lax.broadcasted_iota(jnp.int32, s.shape, 0) + k * _NC
        ik = jnp.min(jnp.where(s == mk, row, _NUM_EMB),
                     axis=0, keepdims=True)
        if k == 0:
            minv, idx = mk, ik
        else:
            take = mk < minv
            idx = jnp.where(take, ik, idx)
            minv = jnp.minimum(minv, mk)

    zq = jnp.zeros((_EMB_DIM, _TS), jnp.float32)
    dn = (((1,), (0,)), ((), ()))
    for k in range(_NCHUNK):
        row = jax.lax.broadcasted_iota(jnp.int32, (_NC, _TS), 0) + k * _NC
        oh = (row == idx).astype(jnp.bfloat16)
        sl = pl.ds(k * _NC, _NC)
        zq = zq + jax.lax.dot_general(cbt_hi_ref[:, sl], oh, dn,
                                      preferred_element_type=jnp.float32)

    zq_ref[0] = zb + (zq - zb)
    idx_ref[0] = idx
    loss_ref[...] += jnp.sum(minv, axis=1, keepdims=True)


def kernel(z_e, codebook):
    B, C, D, H, W = z_e.shape
    S = D * H * W
    z = z_e.reshape(B, C, S)

    cb2 = -2.0 * codebook
    cbt = codebook.T
    cbt_hi = cbt.astype(jnp.bfloat16)
    cbt_lo = (cbt - cbt_hi.astype(jnp.float32)).astype(jnp.bfloat16)

    zq, idx, loss = pl.pallas_call(
        _vq_tile,
        grid=(B, S // _TS),
        in_specs=[
            pl.BlockSpec((1, C, _TS), lambda b, t: (b, 0, t)),
            pl.BlockSpec((_NUM_EMB, _EMB_DIM), lambda b, t: (0, 0)),
            pl.BlockSpec((_EMB_DIM, _NUM_EMB), lambda b, t: (0, 0)),
            pl.BlockSpec((_EMB_DIM, _NUM_EMB), lambda b, t: (0, 0)),
        ],
        out_specs=[
            pl.BlockSpec((1, C, _TS), lambda b, t: (b, 0, t)),
            pl.BlockSpec((1, 1, _TS), lambda b, t: (b, 0, t)),
            pl.BlockSpec((1, 1), lambda b, t: (0, 0)),
        ],
        out_shape=[
            jax.ShapeDtypeStruct((B, C, S), jnp.float32),
            jax.ShapeDtypeStruct((B, 1, S), jnp.int32),
            jax.ShapeDtypeStruct((1, 1), jnp.float32),
        ],
        scratch_shapes=[pltpu.VMEM((_NUM_EMB, _TS), jnp.float32)],
    )(z, cb2, cbt_hi, cbt_lo)

    m = loss[0, 0] / jnp.float32(B * S * C)
    vq_loss = m + jnp.float32(_COMMIT) * m
    return (zq.reshape(B, C, D, H, W), vq_loss,
            idx.reshape(B, D, H, W))
